# Initial kernel scaffold; baseline (speedup 1.0000x reference)
import jax
import jax.numpy as jnp
from jax.experimental import pallas as pl


def kernel(x, edge_index, edge_attr, W1a, B1a, W2a, B2a, W3a, B3a, W4a, B4a, W5a, B5a, G1, Be1, W1b, B1b, W2b, B2b, W3b, B3b, W4b, B4b, W5b, B5b, G2, Be2):
    raise NotImplementedError("write your pallas kernel here")


# speedup vs baseline: 6.4916x; 6.4916x over previous
import functools

import jax
import jax.numpy as jnp
from jax import lax
from jax.experimental import pallas as pl
from jax.experimental.pallas import tpu as pltpu
from jax.experimental.pallas import tpu_sc as plsc

N = 10000
E = 320000
D = 128
DE = 16

NC = 2
NS = 16
NW = NC * NS
EPT = E // NW
CH = 80
NCHUNK = EPT // CH
NPAD = 10240
RPT = NPAD // NS


@functools.cache
def _make_ea_deg_sc():
    mesh = plsc.VectorSubcoreMesh(core_axis_name="c", subcore_axis_name="s")
    return functools.partial(
        pl.kernel,
        mesh=mesh,
        out_type=jax.ShapeDtypeStruct((NC, NPAD, D), jnp.float32),
        scratch_types=[
            pltpu.VMEM((CH,), jnp.int32),
            pltpu.VMEM((CH, D), jnp.float32),
            pltpu.VMEM_SHARED((NPAD, D), jnp.float32),
        ],
    )(_ea_deg_sc_body)


def _ea_deg_sc_body(ea_hbm, dst_hbm, out, didx, rows, acc):
    c = lax.axis_index("c")
    s = lax.axis_index("s")
    tile = c * NS + s

    zeros16 = jnp.zeros((16,), jnp.float32)

    def _zrow(i, _):
        for j in range(D // 16):
            rows[i, pl.ds(j * 16, 16)] = zeros16
        return 0

    lax.fori_loop(0, CH, _zrow, 0)

    def _zchunk(k, _):
        pltpu.sync_copy(rows, acc.at[pl.ds(s * RPT + k * CH, CH)])
        return 0

    lax.fori_loop(0, RPT // CH, _zchunk, 0)
    plsc.subcore_barrier()

    base0 = tile * EPT

    def _body(i, _):
        base = base0 + i * CH
        pltpu.sync_copy(dst_hbm.at[pl.ds(base, CH)], didx)
        pltpu.sync_copy(ea_hbm.at[pl.ds(base, CH)], rows)
        pltpu.sync_copy(rows, acc.at[didx], add=True)
        return 0

    lax.fori_loop(0, NCHUNK, _body, 0)
    plsc.subcore_barrier()

    def _wchunk(k, _):
        rws = pl.ds(s * RPT + k * CH, CH)
        pltpu.sync_copy(acc.at[rws], rows)
        pltpu.sync_copy(rows, out.at[c, rws])
        return 0

    lax.fori_loop(0, RPT // CH, _wchunk, 0)


@functools.cache
def _make_segsum_sc():
    mesh = plsc.VectorSubcoreMesh(core_axis_name="c", subcore_axis_name="s")
    return functools.partial(
        pl.kernel,
        mesh=mesh,
        out_type=jax.ShapeDtypeStruct((NC, NPAD, D), jnp.float32),
        scratch_types=[
            pltpu.VMEM((CH,), jnp.int32),
            pltpu.VMEM((CH,), jnp.int32),
            pltpu.VMEM((CH, D), jnp.float32),
            pltpu.VMEM_SHARED((NPAD, D), jnp.float32),
            pltpu.SemaphoreType.DMA,
        ],
    )(_segsum_sc_body)


def _segsum_sc_body(w2_hbm, src_hbm, dst_hbm, out,
                    sidx, didx, rows, acc, sem):
    c = lax.axis_index("c")
    s = lax.axis_index("s")
    tile = c * NS + s

    zeros16 = jnp.zeros((16,), jnp.float32)

    def _zrow(i, _):
        for j in range(D // 16):
            rows[i, pl.ds(j * 16, 16)] = zeros16
        return 0

    lax.fori_loop(0, CH, _zrow, 0)

    def _zchunk(k, _):
        pltpu.sync_copy(rows, acc.at[pl.ds(s * RPT + k * CH, CH)])
        return 0

    lax.fori_loop(0, RPT // CH, _zchunk, 0)
    plsc.subcore_barrier()

    base0 = tile * EPT

    def _body(i, _):
        base = base0 + i * CH
        pltpu.sync_copy(src_hbm.at[pl.ds(base, CH)], sidx)
        pltpu.sync_copy(dst_hbm.at[pl.ds(base, CH)], didx)
        pltpu.async_copy(w2_hbm.at[sidx], rows, sem).wait()
        pltpu.sync_copy(rows, acc.at[didx], add=True)
        return 0

    lax.fori_loop(0, NCHUNK, _body, 0)
    plsc.subcore_barrier()

    def _wchunk(k, _):
        rws = pl.ds(s * RPT + k * CH, CH)
        pltpu.sync_copy(acc.at[rws], rows)
        pltpu.sync_copy(rows, out.at[c, rws])
        return 0

    lax.fori_loop(0, RPT // CH, _wchunk, 0)


BLK = 1000
GRID = N // BLK

_row_spec = pl.BlockSpec((BLK, D), lambda i: (i, 0))
_p0_spec = pl.BlockSpec((1, BLK, D), lambda i: (0, i, 0))
_p1_spec = pl.BlockSpec((1, BLK, D), lambda i: (1, i, 0))
_w_spec = pl.BlockSpec((D, D), lambda i: (0, 0))
_w5_spec = pl.BlockSpec((DE, D), lambda i: (0, 0))
_b_spec = pl.BlockSpec((1, D), lambda i: (0, 0))


def _mm2_body(x_ref, w1_ref, b1_ref, w2_ref, b2_ref, o1_ref, o2_ref):
    xv = x_ref[...]
    o1_ref[...] = jnp.dot(xv, w1_ref[...], preferred_element_type=jnp.float32) + b1_ref[...]
    o2_ref[...] = jnp.dot(xv, w2_ref[...], preferred_element_type=jnp.float32) + b2_ref[...]


def _mm2(x, W1, B1, W2, B2):
    return pl.pallas_call(
        _mm2_body,
        grid=(GRID,),
        in_specs=[_row_spec, _w_spec, _b_spec, _w_spec, _b_spec],
        out_specs=[_row_spec, _row_spec],
        out_shape=[jax.ShapeDtypeStruct((N, D), jnp.float32)] * 2,
    )(x, W1, B1.reshape(1, D), W2, B2.reshape(1, D))


def _assemble(w1_ref, seg0_ref, seg1_ref, ead0_ref, ead1_ref,
              w5_ref, b5_ref, g_ref, be_ref, res_ref):
    ead = ead0_ref[0] + ead1_ref[0]
    ea = ead[:, 0:DE]
    deg = ead[:, DE:DE + 1]
    z = (w1_ref[...] + seg0_ref[0] + seg1_ref[0]
         + jnp.dot(ea, w5_ref[...], preferred_element_type=jnp.float32)
         + deg * b5_ref[...])
    mu = jnp.mean(z, axis=-1, keepdims=True)
    var = jnp.mean((z - mu) ** 2, axis=-1, keepdims=True)
    zn = (z - mu) / jnp.sqrt(var + 1e-5) * g_ref[...] + be_ref[...]
    return zn + res_ref[...]


def _combine_mm_body(w1_ref, seg0_ref, seg1_ref, ead0_ref, ead1_ref,
                     w5_ref, b5_ref, g_ref, be_ref, res_ref,
                     w1n_ref, b1n_ref, w2n_ref, b2n_ref,
                     z_ref, o1_ref, o2_ref):
    z1 = _assemble(w1_ref, seg0_ref, seg1_ref, ead0_ref, ead1_ref,
                   w5_ref, b5_ref, g_ref, be_ref, res_ref)
    z_ref[...] = z1
    o1_ref[...] = jnp.dot(z1, w1n_ref[...], preferred_element_type=jnp.float32) + b1n_ref[...]
    o2_ref[...] = jnp.dot(z1, w2n_ref[...], preferred_element_type=jnp.float32) + b2n_ref[...]


def _combine_body(w1_ref, seg0_ref, seg1_ref, ead0_ref, ead1_ref,
                  w5_ref, b5_ref, g_ref, be_ref, res_ref, z_ref):
    z_ref[...] = _assemble(w1_ref, seg0_ref, seg1_ref, ead0_ref, ead1_ref,
                           w5_ref, b5_ref, g_ref, be_ref, res_ref)


def _combine_mm(w1, seg, ead, W5, B5, G, Be, res, W1n, B1n, W2n, B2n):
    return pl.pallas_call(
        _combine_mm_body,
        grid=(GRID,),
        in_specs=[_row_spec, _p0_spec, _p1_spec, _p0_spec, _p1_spec,
                  _w5_spec, _b_spec, _b_spec, _b_spec, _row_spec,
                  _w_spec, _b_spec, _w_spec, _b_spec],
        out_specs=[_row_spec, _row_spec, _row_spec],
        out_shape=[jax.ShapeDtypeStruct((N, D), jnp.float32)] * 3,
    )(w1, seg, seg, ead, ead, W5, B5.reshape(1, D),
      G.reshape(1, D), Be.reshape(1, D), res, W1n, B1n.reshape(1, D),
      W2n, B2n.reshape(1, D))


def _combine(w1, seg, ead, W5, B5, G, Be, res):
    return pl.pallas_call(
        _combine_body,
        grid=(GRID,),
        in_specs=[_row_spec, _p0_spec, _p1_spec, _p0_spec, _p1_spec,
                  _w5_spec, _b_spec, _b_spec, _b_spec, _row_spec],
        out_specs=_row_spec,
        out_shape=jax.ShapeDtypeStruct((N, D), jnp.float32),
    )(w1, seg, seg, ead, ead, W5, B5.reshape(1, D),
      G.reshape(1, D), Be.reshape(1, D), res)


def kernel(x, edge_index, edge_attr,
           W1a, B1a, W2a, B2a, W3a, B3a, W4a, B4a, W5a, B5a, G1, Be1,
           W1b, B1b, W2b, B2b, W3b, B3b, W4b, B4b, W5b, B5b, G2, Be2):
    dst = edge_index[0]
    src = edge_index[1]

    ea128 = jnp.pad(
        jnp.concatenate([edge_attr, jnp.ones((E, 1), jnp.float32)], axis=1),
        ((0, 0), (0, D - DE - 1)))

    ead = _make_ea_deg_sc()(ea128, dst)
    w1a, w2a = _mm2(x, W1a, B1a, W2a, B2a)
    sega = _make_segsum_sc()(w2a, src, dst)
    z1, w1b, w2b = _combine_mm(w1a, sega, ead, W5a, B5a, G1, Be1, x,
                               W1b, B1b, W2b, B2b)
    segb = _make_segsum_sc()(w2b, src, dst)
    z2 = _combine(w1b, segb, ead, W5b, B5b, G2, Be2, z1)
    return z2


# speedup vs baseline: 13.1629x; 2.0277x over previous
import functools

import jax
import jax.numpy as jnp
from jax import lax
from jax.experimental import pallas as pl
from jax.experimental.pallas import tpu as pltpu
from jax.experimental.pallas import tpu_sc as plsc

N = 10000
E = 320000
D = 128
DE = 16

NC = 2
NS = 16
NW = NC * NS
EPT = E // NW
CH = 80
NCHUNK = EPT // CH
NPAD = 10240
RPT = NPAD // NS


@functools.cache
def _make_ea_deg_sc():
    mesh = plsc.VectorSubcoreMesh(core_axis_name="c", subcore_axis_name="s")
    return functools.partial(
        pl.kernel,
        mesh=mesh,
        out_type=jax.ShapeDtypeStruct((NC, NPAD, D), jnp.float32),
        scratch_types=[
            pltpu.VMEM((CH,), jnp.int32),
            pltpu.VMEM((CH,), jnp.int32),
            pltpu.VMEM((CH, D), jnp.float32),
            pltpu.VMEM((CH, D), jnp.float32),
            pltpu.VMEM((CH, D), jnp.float32),
            pltpu.VMEM_SHARED((NPAD, D), jnp.float32),
            pltpu.SemaphoreType.DMA,
            pltpu.SemaphoreType.DMA,
            pltpu.SemaphoreType.DMA,
            pltpu.SemaphoreType.DMA,
        ],
    )(_ea_deg_sc_body)


def _ea_deg_sc_body(ea_hbm, dst_hbm, out,
                    didx_a, didx_b, rows_a, rows_b, zbuf, acc,
                    dsem_a, dsem_b, gsem_a, gsem_b):
    c = lax.axis_index("c")
    s = lax.axis_index("s")
    tile = c * NS + s

    zeros16 = jnp.zeros((16,), jnp.float32)

    def _zrow(i, _):
        for j in range(D // 16):
            zbuf[i, pl.ds(j * 16, 16)] = zeros16
        return 0

    lax.fori_loop(0, CH, _zrow, 0)

    def _zchunk(k, _):
        pltpu.sync_copy(zbuf, acc.at[pl.ds(s * RPT + k * CH, CH)])
        return 0

    lax.fori_loop(0, RPT // CH, _zchunk, 0)
    plsc.subcore_barrier()

    base0 = tile * EPT

    def _start(j, didx, rows, dsem, gsem):
        pltpu.async_copy(dst_hbm.at[pl.ds(base0 + j * CH, CH)], didx, dsem)
        pltpu.async_copy(ea_hbm.at[pl.ds(base0 + j * CH, CH)], rows, gsem)

    def _finish(didx, rows, dsem, gsem):
        pltpu.make_async_copy(dst_hbm.at[pl.ds(base0, CH)], didx, dsem).wait()
        pltpu.make_async_copy(ea_hbm.at[pl.ds(base0, CH)], rows, gsem).wait()
        pltpu.sync_copy(rows, acc.at[didx], add=True)

    _start(0, didx_a, rows_a, dsem_a, gsem_a)
    _start(1, didx_b, rows_b, dsem_b, gsem_b)

    def _body(k, _):
        i = 2 * k
        _finish(didx_a, rows_a, dsem_a, gsem_a)
        _start(i + 2, didx_a, rows_a, dsem_a, gsem_a)
        _finish(didx_b, rows_b, dsem_b, gsem_b)
        _start(i + 3, didx_b, rows_b, dsem_b, gsem_b)
        return 0

    lax.fori_loop(0, (NCHUNK - 3) // 2, _body, 0)
    _finish(didx_a, rows_a, dsem_a, gsem_a)
    _start(NCHUNK - 1, didx_a, rows_a, dsem_a, gsem_a)
    _finish(didx_b, rows_b, dsem_b, gsem_b)
    _finish(didx_a, rows_a, dsem_a, gsem_a)
    plsc.subcore_barrier()

    def _wchunk(k, _):
        rws = pl.ds(s * RPT + k * CH, CH)
        pltpu.sync_copy(acc.at[rws], zbuf)
        pltpu.sync_copy(zbuf, out.at[c, rws])
        return 0

    lax.fori_loop(0, RPT // CH, _wchunk, 0)


@functools.cache
def _make_segsum_sc():
    mesh = plsc.VectorSubcoreMesh(core_axis_name="c", subcore_axis_name="s")
    return functools.partial(
        pl.kernel,
        mesh=mesh,
        out_type=jax.ShapeDtypeStruct((NC, NPAD, D), jnp.float32),
        scratch_types=[
            pltpu.VMEM((EPT,), jnp.int32),
            pltpu.VMEM((CH,), jnp.int32),
            pltpu.VMEM((CH,), jnp.int32),
            pltpu.VMEM((CH, D), jnp.float32),
            pltpu.VMEM((CH, D), jnp.float32),
            pltpu.VMEM((CH, D), jnp.float32),
            pltpu.VMEM_SHARED((NPAD, D), jnp.float32),
            pltpu.SemaphoreType.DMA,
            pltpu.SemaphoreType.DMA,
            pltpu.SemaphoreType.DMA,
            pltpu.SemaphoreType.DMA,
        ],
    )(_segsum_sc_body)


def _segsum_sc_body(w2_hbm, src_hbm, dst_hbm, out,
                    sidx_all, didx_a, didx_b, rows_a, rows_b, zbuf, acc,
                    dsem_a, dsem_b, gsem_a, gsem_b):
    c = lax.axis_index("c")
    s = lax.axis_index("s")
    tile = c * NS + s

    zeros16 = jnp.zeros((16,), jnp.float32)

    def _zrow(i, _):
        for j in range(D // 16):
            zbuf[i, pl.ds(j * 16, 16)] = zeros16
        return 0

    lax.fori_loop(0, CH, _zrow, 0)

    def _zchunk(k, _):
        pltpu.sync_copy(zbuf, acc.at[pl.ds(s * RPT + k * CH, CH)])
        return 0

    lax.fori_loop(0, RPT // CH, _zchunk, 0)

    base0 = tile * EPT
    pltpu.sync_copy(src_hbm.at[pl.ds(base0, EPT)], sidx_all)
    plsc.subcore_barrier()

    def _start(j, didx, rows, dsem, gsem):
        pltpu.async_copy(dst_hbm.at[pl.ds(base0 + j * CH, CH)], didx, dsem)
        pltpu.async_copy(w2_hbm.at[sidx_all.at[pl.ds(j * CH, CH)]], rows, gsem)

    def _finish(didx, rows, dsem, gsem):
        pltpu.make_async_copy(dst_hbm.at[pl.ds(base0, CH)], didx, dsem).wait()
        pltpu.make_async_copy(
            w2_hbm.at[sidx_all.at[pl.ds(0, CH)]], rows, gsem).wait()
        pltpu.sync_copy(rows, acc.at[didx], add=True)

    _start(0, didx_a, rows_a, dsem_a, gsem_a)
    _start(1, didx_b, rows_b, dsem_b, gsem_b)

    def _body(k, _):
        i = 2 * k
        _finish(didx_a, rows_a, dsem_a, gsem_a)
        _start(i + 2, didx_a, rows_a, dsem_a, gsem_a)
        _finish(didx_b, rows_b, dsem_b, gsem_b)
        _start(i + 3, didx_b, rows_b, dsem_b, gsem_b)
        return 0

    lax.fori_loop(0, (NCHUNK - 3) // 2, _body, 0)
    _finish(didx_a, rows_a, dsem_a, gsem_a)
    _start(NCHUNK - 1, didx_a, rows_a, dsem_a, gsem_a)
    _finish(didx_b, rows_b, dsem_b, gsem_b)
    _finish(didx_a, rows_a, dsem_a, gsem_a)
    plsc.subcore_barrier()

    def _wchunk(k, _):
        rws = pl.ds(s * RPT + k * CH, CH)
        pltpu.sync_copy(acc.at[rws], zbuf)
        pltpu.sync_copy(zbuf, out.at[c, rws])
        return 0

    lax.fori_loop(0, RPT // CH, _wchunk, 0)


BLK = 1000
GRID = N // BLK

_row_spec = pl.BlockSpec((BLK, D), lambda i: (i, 0))
_p0_spec = pl.BlockSpec((1, BLK, D), lambda i: (0, i, 0))
_p1_spec = pl.BlockSpec((1, BLK, D), lambda i: (1, i, 0))
_w_spec = pl.BlockSpec((D, D), lambda i: (0, 0))
_w5_spec = pl.BlockSpec((DE, D), lambda i: (0, 0))
_b_spec = pl.BlockSpec((1, D), lambda i: (0, 0))


def _mm2_body(x_ref, w1_ref, b1_ref, w2_ref, b2_ref, o1_ref, o2_ref):
    xv = x_ref[...]
    o1_ref[...] = jnp.dot(xv, w1_ref[...], preferred_element_type=jnp.float32) + b1_ref[...]
    o2_ref[...] = jnp.dot(xv, w2_ref[...], preferred_element_type=jnp.float32) + b2_ref[...]


def _mm2(x, W1, B1, W2, B2):
    return pl.pallas_call(
        _mm2_body,
        grid=(GRID,),
        in_specs=[_row_spec, _w_spec, _b_spec, _w_spec, _b_spec],
        out_specs=[_row_spec, _row_spec],
        out_shape=[jax.ShapeDtypeStruct((N, D), jnp.float32)] * 2,
    )(x, W1, B1.reshape(1, D), W2, B2.reshape(1, D))


def _assemble(w1_ref, seg0_ref, seg1_ref, ead0_ref, ead1_ref,
              w5_ref, b5_ref, g_ref, be_ref, res_ref):
    ead = ead0_ref[0] + ead1_ref[0]
    ea = ead[:, 0:DE]
    deg = ead[:, DE:DE + 1]
    z = (w1_ref[...] + seg0_ref[0] + seg1_ref[0]
         + jnp.dot(ea, w5_ref[...], preferred_element_type=jnp.float32)
         + deg * b5_ref[...])
    mu = jnp.mean(z, axis=-1, keepdims=True)
    var = jnp.mean((z - mu) ** 2, axis=-1, keepdims=True)
    zn = (z - mu) / jnp.sqrt(var + 1e-5) * g_ref[...] + be_ref[...]
    return zn + res_ref[...]


def _combine_mm_body(w1_ref, seg0_ref, seg1_ref, ead0_ref, ead1_ref,
                     w5_ref, b5_ref, g_ref, be_ref, res_ref,
                     w1n_ref, b1n_ref, w2n_ref, b2n_ref,
                     z_ref, o1_ref, o2_ref):
    z1 = _assemble(w1_ref, seg0_ref, seg1_ref, ead0_ref, ead1_ref,
                   w5_ref, b5_ref, g_ref, be_ref, res_ref)
    z_ref[...] = z1
    o1_ref[...] = jnp.dot(z1, w1n_ref[...], preferred_element_type=jnp.float32) + b1n_ref[...]
    o2_ref[...] = jnp.dot(z1, w2n_ref[...], preferred_element_type=jnp.float32) + b2n_ref[...]


def _combine_body(w1_ref, seg0_ref, seg1_ref, ead0_ref, ead1_ref,
                  w5_ref, b5_ref, g_ref, be_ref, res_ref, z_ref):
    z_ref[...] = _assemble(w1_ref, seg0_ref, seg1_ref, ead0_ref, ead1_ref,
                           w5_ref, b5_ref, g_ref, be_ref, res_ref)


def _combine_mm(w1, seg, ead, W5, B5, G, Be, res, W1n, B1n, W2n, B2n):
    return pl.pallas_call(
        _combine_mm_body,
        grid=(GRID,),
        in_specs=[_row_spec, _p0_spec, _p1_spec, _p0_spec, _p1_spec,
                  _w5_spec, _b_spec, _b_spec, _b_spec, _row_spec,
                  _w_spec, _b_spec, _w_spec, _b_spec],
        out_specs=[_row_spec, _row_spec, _row_spec],
        out_shape=[jax.ShapeDtypeStruct((N, D), jnp.float32)] * 3,
    )(w1, seg, seg, ead, ead, W5, B5.reshape(1, D),
      G.reshape(1, D), Be.reshape(1, D), res, W1n, B1n.reshape(1, D),
      W2n, B2n.reshape(1, D))


def _combine(w1, seg, ead, W5, B5, G, Be, res):
    return pl.pallas_call(
        _combine_body,
        grid=(GRID,),
        in_specs=[_row_spec, _p0_spec, _p1_spec, _p0_spec, _p1_spec,
                  _w5_spec, _b_spec, _b_spec, _b_spec, _row_spec],
        out_specs=_row_spec,
        out_shape=jax.ShapeDtypeStruct((N, D), jnp.float32),
    )(w1, seg, seg, ead, ead, W5, B5.reshape(1, D),
      G.reshape(1, D), Be.reshape(1, D), res)


def kernel(x, edge_index, edge_attr,
           W1a, B1a, W2a, B2a, W3a, B3a, W4a, B4a, W5a, B5a, G1, Be1,
           W1b, B1b, W2b, B2b, W3b, B3b, W4b, B4b, W5b, B5b, G2, Be2):
    dst = edge_index[0]
    src = edge_index[1]

    ea128 = jnp.pad(
        jnp.concatenate([edge_attr, jnp.ones((E, 1), jnp.float32)], axis=1),
        ((0, 0), (0, D - DE - 1)))

    ead = _make_ea_deg_sc()(ea128, dst)
    w1a, w2a = _mm2(x, W1a, B1a, W2a, B2a)
    sega = _make_segsum_sc()(w2a, src, dst)
    z1, w1b, w2b = _combine_mm(w1a, sega, ead, W5a, B5a, G1, Be1, x,
                               W1b, B1b, W2b, B2b)
    segb = _make_segsum_sc()(w2b, src, dst)
    z2 = _combine(w1b, segb, ead, W5b, B5b, G2, Be2, z1)
    return z2


# speedup vs baseline: 14.6585x; 1.1136x over previous
import functools

import jax
import jax.numpy as jnp
from jax import lax
from jax.experimental import pallas as pl
from jax.experimental.pallas import tpu as pltpu
from jax.experimental.pallas import tpu_sc as plsc

N = 10000
E = 320000
D = 128
DE = 16

NC = 2
NS = 16
NW = NC * NS
EPT = E // NW
CH = 80
NCHUNK = EPT // CH
NPAD = 10240
RPT = NPAD // NS
NBUF = 3


def _sw_pipeline(nchunk, nbuf, start, finish):
    for b in range(nbuf):
        start(b, b)
    full = nchunk // nbuf - 1

    def _body(k, _):
        for p in range(nbuf):
            finish(p)
            start(nbuf * (k + 1) + p, p)
        return 0

    lax.fori_loop(0, full, _body, 0)
    done = nbuf * full
    for t in range(nchunk - done):
        j = done + t
        finish(j % nbuf)
        nxt = done + nbuf + t
        if nxt < nchunk:
            start(nxt, j % nbuf)


@functools.cache
def _make_ea_deg_sc():
    mesh = plsc.VectorSubcoreMesh(core_axis_name="c", subcore_axis_name="s")
    return functools.partial(
        pl.kernel,
        mesh=mesh,
        out_type=jax.ShapeDtypeStruct((NC, NPAD, D), jnp.float32),
        scratch_types=(
            [pltpu.VMEM((CH,), jnp.int32) for _ in range(NBUF)]
            + [pltpu.VMEM((CH, D), jnp.float32) for _ in range(NBUF)]
            + [pltpu.VMEM_SHARED((NPAD, D), jnp.float32)]
            + [pltpu.SemaphoreType.DMA for _ in range(2 * NBUF)]
        ),
    )(_ea_deg_sc_body)


def _ea_deg_sc_body(ea_hbm, dst_hbm, out, *refs):
    didxs = refs[0:NBUF]
    rows = refs[NBUF:2 * NBUF]
    acc = refs[2 * NBUF]
    dsems = refs[2 * NBUF + 1:2 * NBUF + 1 + NBUF]
    gsems = refs[2 * NBUF + 1 + NBUF:2 * NBUF + 1 + 2 * NBUF]
    c = lax.axis_index("c")
    s = lax.axis_index("s")
    tile = c * NS + s

    zeros16 = jnp.zeros((16,), jnp.float32)
    zbuf = rows[0]

    def _zrow(i, _):
        for j in range(D // 16):
            zbuf[i, pl.ds(j * 16, 16)] = zeros16
        return 0

    lax.fori_loop(0, CH, _zrow, 0)

    def _zchunk(k, _):
        pltpu.sync_copy(zbuf, acc.at[pl.ds(s * RPT + k * CH, CH)])
        return 0

    lax.fori_loop(0, RPT // CH, _zchunk, 0)
    plsc.subcore_barrier()

    base0 = tile * EPT

    def _start(j, p):
        pltpu.async_copy(dst_hbm.at[pl.ds(base0 + j * CH, CH)], didxs[p],
                         dsems[p])
        pltpu.async_copy(ea_hbm.at[pl.ds(base0 + j * CH, CH)], rows[p],
                         gsems[p])

    def _finish(p):
        pltpu.make_async_copy(dst_hbm.at[pl.ds(base0, CH)], didxs[p],
                              dsems[p]).wait()
        pltpu.make_async_copy(ea_hbm.at[pl.ds(base0, CH)], rows[p],
                              gsems[p]).wait()
        pltpu.sync_copy(rows[p], acc.at[didxs[p]], add=True)

    _sw_pipeline(NCHUNK, NBUF, _start, _finish)
    plsc.subcore_barrier()

    wbuf = rows[0]

    def _wchunk(k, _):
        rws = pl.ds(s * RPT + k * CH, CH)
        pltpu.sync_copy(acc.at[rws], wbuf)
        pltpu.sync_copy(wbuf, out.at[c, rws])
        return 0

    lax.fori_loop(0, RPT // CH, _wchunk, 0)


@functools.cache
def _make_segsum_sc():
    mesh = plsc.VectorSubcoreMesh(core_axis_name="c", subcore_axis_name="s")
    return functools.partial(
        pl.kernel,
        mesh=mesh,
        out_type=jax.ShapeDtypeStruct((NC, NPAD, D), jnp.float32),
        scratch_types=(
            [pltpu.VMEM((EPT,), jnp.int32)]
            + [pltpu.VMEM((CH,), jnp.int32) for _ in range(NBUF)]
            + [pltpu.VMEM((CH, D), jnp.float32) for _ in range(NBUF)]
            + [pltpu.VMEM_SHARED((NPAD, D), jnp.float32)]
            + [pltpu.SemaphoreType.DMA for _ in range(2 * NBUF)]
        ),
    )(_segsum_sc_body)


def _segsum_sc_body(w2_hbm, src_hbm, dst_hbm, out, sidx_all, *refs):
    didxs = refs[0:NBUF]
    rows = refs[NBUF:2 * NBUF]
    acc = refs[2 * NBUF]
    dsems = refs[2 * NBUF + 1:2 * NBUF + 1 + NBUF]
    gsems = refs[2 * NBUF + 1 + NBUF:2 * NBUF + 1 + 2 * NBUF]
    c = lax.axis_index("c")
    s = lax.axis_index("s")
    tile = c * NS + s

    zeros16 = jnp.zeros((16,), jnp.float32)
    zbuf = rows[0]

    def _zrow(i, _):
        for j in range(D // 16):
            zbuf[i, pl.ds(j * 16, 16)] = zeros16
        return 0

    lax.fori_loop(0, CH, _zrow, 0)

    def _zchunk(k, _):
        pltpu.sync_copy(zbuf, acc.at[pl.ds(s * RPT + k * CH, CH)])
        return 0

    lax.fori_loop(0, RPT // CH, _zchunk, 0)

    base0 = tile * EPT
    pltpu.sync_copy(src_hbm.at[pl.ds(base0, EPT)], sidx_all)
    plsc.subcore_barrier()

    def _start(j, p):
        pltpu.async_copy(dst_hbm.at[pl.ds(base0 + j * CH, CH)], didxs[p],
                         dsems[p])
        pltpu.async_copy(w2_hbm.at[sidx_all.at[pl.ds(j * CH, CH)]], rows[p],
                         gsems[p])

    def _finish(p):
        pltpu.make_async_copy(dst_hbm.at[pl.ds(base0, CH)], didxs[p],
                              dsems[p]).wait()
        pltpu.make_async_copy(
            w2_hbm.at[sidx_all.at[pl.ds(0, CH)]], rows[p], gsems[p]).wait()
        pltpu.sync_copy(rows[p], acc.at[didxs[p]], add=True)

    _sw_pipeline(NCHUNK, NBUF, _start, _finish)
    plsc.subcore_barrier()

    wbuf = rows[0]

    def _wchunk(k, _):
        rws = pl.ds(s * RPT + k * CH, CH)
        pltpu.sync_copy(acc.at[rws], wbuf)
        pltpu.sync_copy(wbuf, out.at[c, rws])
        return 0

    lax.fori_loop(0, RPT // CH, _wchunk, 0)


BLK = 1000
GRID = N // BLK

_row_spec = pl.BlockSpec((BLK, D), lambda i: (i, 0))
_p0_spec = pl.BlockSpec((1, BLK, D), lambda i: (0, i, 0))
_p1_spec = pl.BlockSpec((1, BLK, D), lambda i: (1, i, 0))
_w_spec = pl.BlockSpec((D, D), lambda i: (0, 0))
_w5_spec = pl.BlockSpec((DE, D), lambda i: (0, 0))
_b_spec = pl.BlockSpec((1, D), lambda i: (0, 0))


def _mm2_body(x_ref, w1_ref, b1_ref, w2_ref, b2_ref, o1_ref, o2_ref):
    xv = x_ref[...]
    o1_ref[...] = jnp.dot(xv, w1_ref[...], preferred_element_type=jnp.float32) + b1_ref[...]
    o2_ref[...] = jnp.dot(xv, w2_ref[...], preferred_element_type=jnp.float32) + b2_ref[...]


def _mm2(x, W1, B1, W2, B2):
    return pl.pallas_call(
        _mm2_body,
        grid=(GRID,),
        in_specs=[_row_spec, _w_spec, _b_spec, _w_spec, _b_spec],
        out_specs=[_row_spec, _row_spec],
        out_shape=[jax.ShapeDtypeStruct((N, D), jnp.float32)] * 2,
    )(x, W1, B1.reshape(1, D), W2, B2.reshape(1, D))


def _assemble(w1_ref, seg0_ref, seg1_ref, ead0_ref, ead1_ref,
              w5_ref, b5_ref, g_ref, be_ref, res_ref):
    ead = ead0_ref[0] + ead1_ref[0]
    ea = ead[:, 0:DE]
    deg = ead[:, DE:DE + 1]
    z = (w1_ref[...] + seg0_ref[0] + seg1_ref[0]
         + jnp.dot(ea, w5_ref[...], preferred_element_type=jnp.float32)
         + deg * b5_ref[...])
    mu = jnp.mean(z, axis=-1, keepdims=True)
    var = jnp.mean((z - mu) ** 2, axis=-1, keepdims=True)
    zn = (z - mu) / jnp.sqrt(var + 1e-5) * g_ref[...] + be_ref[...]
    return zn + res_ref[...]


def _combine_mm_body(w1_ref, seg0_ref, seg1_ref, ead0_ref, ead1_ref,
                     w5_ref, b5_ref, g_ref, be_ref, res_ref,
                     w1n_ref, b1n_ref, w2n_ref, b2n_ref,
                     z_ref, o1_ref, o2_ref):
    z1 = _assemble(w1_ref, seg0_ref, seg1_ref, ead0_ref, ead1_ref,
                   w5_ref, b5_ref, g_ref, be_ref, res_ref)
    z_ref[...] = z1
    o1_ref[...] = jnp.dot(z1, w1n_ref[...], preferred_element_type=jnp.float32) + b1n_ref[...]
    o2_ref[...] = jnp.dot(z1, w2n_ref[...], preferred_element_type=jnp.float32) + b2n_ref[...]


def _combine_body(w1_ref, seg0_ref, seg1_ref, ead0_ref, ead1_ref,
                  w5_ref, b5_ref, g_ref, be_ref, res_ref, z_ref):
    z_ref[...] = _assemble(w1_ref, seg0_ref, seg1_ref, ead0_ref, ead1_ref,
                           w5_ref, b5_ref, g_ref, be_ref, res_ref)


def _combine_mm(w1, seg, ead, W5, B5, G, Be, res, W1n, B1n, W2n, B2n):
    return pl.pallas_call(
        _combine_mm_body,
        grid=(GRID,),
        in_specs=[_row_spec, _p0_spec, _p1_spec, _p0_spec, _p1_spec,
                  _w5_spec, _b_spec, _b_spec, _b_spec, _row_spec,
                  _w_spec, _b_spec, _w_spec, _b_spec],
        out_specs=[_row_spec, _row_spec, _row_spec],
        out_shape=[jax.ShapeDtypeStruct((N, D), jnp.float32)] * 3,
    )(w1, seg, seg, ead, ead, W5, B5.reshape(1, D),
      G.reshape(1, D), Be.reshape(1, D), res, W1n, B1n.reshape(1, D),
      W2n, B2n.reshape(1, D))


def _combine(w1, seg, ead, W5, B5, G, Be, res):
    return pl.pallas_call(
        _combine_body,
        grid=(GRID,),
        in_specs=[_row_spec, _p0_spec, _p1_spec, _p0_spec, _p1_spec,
                  _w5_spec, _b_spec, _b_spec, _b_spec, _row_spec],
        out_specs=_row_spec,
        out_shape=jax.ShapeDtypeStruct((N, D), jnp.float32),
    )(w1, seg, seg, ead, ead, W5, B5.reshape(1, D),
      G.reshape(1, D), Be.reshape(1, D), res)


def kernel(x, edge_index, edge_attr,
           W1a, B1a, W2a, B2a, W3a, B3a, W4a, B4a, W5a, B5a, G1, Be1,
           W1b, B1b, W2b, B2b, W3b, B3b, W4b, B4b, W5b, B5b, G2, Be2):
    dst = edge_index[0]
    src = edge_index[1]

    ea128 = jnp.pad(
        jnp.concatenate([edge_attr, jnp.ones((E, 1), jnp.float32)], axis=1),
        ((0, 0), (0, D - DE - 1)))

    ead = _make_ea_deg_sc()(ea128, dst)
    w1a, w2a = _mm2(x, W1a, B1a, W2a, B2a)
    sega = _make_segsum_sc()(w2a, src, dst)
    z1, w1b, w2b = _combine_mm(w1a, sega, ead, W5a, B5a, G1, Be1, x,
                               W1b, B1b, W2b, B2b)
    segb = _make_segsum_sc()(w2b, src, dst)
    z2 = _combine(w1b, segb, ead, W5b, B5b, G2, Be2, z1)
    return z2


# speedup vs baseline: 14.6773x; 1.0013x over previous
import functools

import jax
import jax.numpy as jnp
from jax import lax
from jax.experimental import pallas as pl
from jax.experimental.pallas import tpu as pltpu
from jax.experimental.pallas import tpu_sc as plsc

N = 10000
E = 320000
D = 128
DE = 16

NC = 2
NS = 16
NW = NC * NS
EPT = E // NW
CH = 80
NCHUNK = EPT // CH
NPAD = 10240
RPT = NPAD // NS
NBUF = 3
DEW = 32


def _sw_pipeline(nchunk, nbuf, start, finish):
    for b in range(nbuf):
        start(b, b)
    full = nchunk // nbuf - 1

    def _body(k, _):
        for p in range(nbuf):
            finish(p)
            start(nbuf * (k + 1) + p, p)
        return 0

    lax.fori_loop(0, full, _body, 0)
    done = nbuf * full
    for t in range(nchunk - done):
        j = done + t
        finish(j % nbuf)
        nxt = done + nbuf + t
        if nxt < nchunk:
            start(nxt, j % nbuf)


@functools.cache
def _make_ea_deg_sc():
    mesh = plsc.VectorSubcoreMesh(core_axis_name="c", subcore_axis_name="s")
    return functools.partial(
        pl.kernel,
        mesh=mesh,
        out_type=jax.ShapeDtypeStruct((NC, NPAD, D), jnp.float32),
        scratch_types=(
            [pltpu.VMEM((CH,), jnp.int32) for _ in range(NBUF)]
            + [pltpu.VMEM((CH, D), jnp.float32) for _ in range(NBUF)]
            + [pltpu.VMEM_SHARED((NPAD, D), jnp.float32)]
            + [pltpu.SemaphoreType.DMA for _ in range(2 * NBUF)]
        ),
    )(_ea_deg_sc_body)


def _ea_deg_sc_body(ea_hbm, dst_hbm, out, *refs):
    didxs = refs[0:NBUF]
    rows = refs[NBUF:2 * NBUF]
    acc = refs[2 * NBUF]
    dsems = refs[2 * NBUF + 1:2 * NBUF + 1 + NBUF]
    gsems = refs[2 * NBUF + 1 + NBUF:2 * NBUF + 1 + 2 * NBUF]
    c = lax.axis_index("c")
    s = lax.axis_index("s")
    tile = c * NS + s

    zeros16 = jnp.zeros((16,), jnp.float32)
    zbuf = rows[0]

    def _zrow(i, _):
        for j in range(D // 16):
            zbuf[i, pl.ds(j * 16, 16)] = zeros16
        return 0

    lax.fori_loop(0, CH, _zrow, 0)

    def _zchunk(k, _):
        pltpu.sync_copy(zbuf, acc.at[pl.ds(s * RPT + k * CH, CH)])
        return 0

    lax.fori_loop(0, RPT // CH, _zchunk, 0)
    plsc.subcore_barrier()

    base0 = tile * EPT

    def _start(j, p):
        pltpu.async_copy(dst_hbm.at[pl.ds(base0 + j * CH, CH)], didxs[p],
                         dsems[p])
        pltpu.async_copy(ea_hbm.at[pl.ds(base0 + j * CH, CH)], rows[p],
                         gsems[p])

    def _finish(p):
        pltpu.make_async_copy(dst_hbm.at[pl.ds(base0, CH)], didxs[p],
                              dsems[p]).wait()
        pltpu.make_async_copy(ea_hbm.at[pl.ds(base0, CH)], rows[p],
                              gsems[p]).wait()
        pltpu.sync_copy(rows[p], acc.at[didxs[p]], add=True)

    _sw_pipeline(NCHUNK, NBUF, _start, _finish)
    plsc.subcore_barrier()

    wbuf = rows[0]

    def _wchunk(k, _):
        rws = pl.ds(s * RPT + k * CH, CH)
        pltpu.sync_copy(acc.at[rws], wbuf)
        pltpu.sync_copy(wbuf, out.at[c, rws])
        return 0

    lax.fori_loop(0, RPT // CH, _wchunk, 0)


@functools.cache
def _make_segsum_sc():
    mesh = plsc.VectorSubcoreMesh(core_axis_name="c", subcore_axis_name="s")
    return functools.partial(
        pl.kernel,
        mesh=mesh,
        out_type=jax.ShapeDtypeStruct((NC, NPAD, D), jnp.float32),
        scratch_types=(
            [pltpu.VMEM((EPT,), jnp.int32)]
            + [pltpu.VMEM((CH,), jnp.int32) for _ in range(NBUF)]
            + [pltpu.VMEM((CH, D), jnp.float32) for _ in range(NBUF)]
            + [pltpu.VMEM_SHARED((NPAD, D), jnp.float32)]
            + [pltpu.SemaphoreType.DMA for _ in range(2 * NBUF)]
        ),
    )(_segsum_sc_body)


def _segsum_sc_body(w2_hbm, src_hbm, dst_hbm, out, sidx_all, *refs):
    didxs = refs[0:NBUF]
    rows = refs[NBUF:2 * NBUF]
    acc = refs[2 * NBUF]
    dsems = refs[2 * NBUF + 1:2 * NBUF + 1 + NBUF]
    gsems = refs[2 * NBUF + 1 + NBUF:2 * NBUF + 1 + 2 * NBUF]
    c = lax.axis_index("c")
    s = lax.axis_index("s")
    tile = c * NS + s

    zeros16 = jnp.zeros((16,), jnp.float32)
    zbuf = rows[0]

    def _zrow(i, _):
        for j in range(D // 16):
            zbuf[i, pl.ds(j * 16, 16)] = zeros16
        return 0

    lax.fori_loop(0, CH, _zrow, 0)

    def _zchunk(k, _):
        pltpu.sync_copy(zbuf, acc.at[pl.ds(s * RPT + k * CH, CH)])
        return 0

    lax.fori_loop(0, RPT // CH, _zchunk, 0)

    base0 = tile * EPT
    pltpu.sync_copy(src_hbm.at[pl.ds(base0, EPT)], sidx_all)
    plsc.subcore_barrier()

    def _start(j, p):
        pltpu.async_copy(dst_hbm.at[pl.ds(base0 + j * CH, CH)], didxs[p],
                         dsems[p])
        pltpu.async_copy(w2_hbm.at[sidx_all.at[pl.ds(j * CH, CH)]], rows[p],
                         gsems[p])

    def _finish(p):
        pltpu.make_async_copy(dst_hbm.at[pl.ds(base0, CH)], didxs[p],
                              dsems[p]).wait()
        pltpu.make_async_copy(
            w2_hbm.at[sidx_all.at[pl.ds(0, CH)]], rows[p], gsems[p]).wait()
        pltpu.sync_copy(rows[p], acc.at[didxs[p]], add=True)

    _sw_pipeline(NCHUNK, NBUF, _start, _finish)
    plsc.subcore_barrier()

    wbuf = rows[0]

    def _wchunk(k, _):
        rws = pl.ds(s * RPT + k * CH, CH)
        pltpu.sync_copy(acc.at[rws], wbuf)
        pltpu.sync_copy(wbuf, out.at[c, rws])
        return 0

    lax.fori_loop(0, RPT // CH, _wchunk, 0)


BLK = 1000
GRID = N // BLK

_row_spec = pl.BlockSpec((BLK, D), lambda i: (i, 0))
_p0_spec = pl.BlockSpec((1, BLK, D), lambda i: (0, i, 0))
_p1_spec = pl.BlockSpec((1, BLK, D), lambda i: (1, i, 0))
_e0_spec = pl.BlockSpec((1, BLK, DEW), lambda i: (0, i, 0))
_e1_spec = pl.BlockSpec((1, BLK, DEW), lambda i: (1, i, 0))
_w_spec = pl.BlockSpec((D, D), lambda i: (0, 0))
_w5_spec = pl.BlockSpec((DE, D), lambda i: (0, 0))
_b_spec = pl.BlockSpec((1, D), lambda i: (0, 0))


def _mm2_body(x_ref, w1_ref, b1_ref, w2_ref, b2_ref, o1_ref, o2_ref):
    xv = x_ref[...]
    o1_ref[...] = jnp.dot(xv, w1_ref[...], preferred_element_type=jnp.float32) + b1_ref[...]
    o2_ref[...] = jnp.dot(xv, w2_ref[...], preferred_element_type=jnp.float32) + b2_ref[...]


def _mm2(x, W1, B1, W2, B2):
    return pl.pallas_call(
        _mm2_body,
        grid=(GRID,),
        in_specs=[_row_spec, _w_spec, _b_spec, _w_spec, _b_spec],
        out_specs=[_row_spec, _row_spec],
        out_shape=[jax.ShapeDtypeStruct((N, D), jnp.float32)] * 2,
    )(x, W1, B1.reshape(1, D), W2, B2.reshape(1, D))


def _assemble(w1_ref, seg0_ref, seg1_ref, ead0_ref, ead1_ref,
              w5_ref, b5_ref, g_ref, be_ref, res_ref):
    ead = ead0_ref[0] + ead1_ref[0]
    ea = ead[:, 0:DE]
    deg = ead[:, DE:DE + 1]
    z = (w1_ref[...] + seg0_ref[0] + seg1_ref[0]
         + jnp.dot(ea, w5_ref[...], preferred_element_type=jnp.float32)
         + deg * b5_ref[...])
    mu = jnp.mean(z, axis=-1, keepdims=True)
    var = jnp.mean((z - mu) ** 2, axis=-1, keepdims=True)
    zn = (z - mu) / jnp.sqrt(var + 1e-5) * g_ref[...] + be_ref[...]
    return zn + res_ref[...]


def _combine_mm_body(w1_ref, seg0_ref, seg1_ref, ead0_ref, ead1_ref,
                     w5_ref, b5_ref, g_ref, be_ref, res_ref,
                     w1n_ref, b1n_ref, w2n_ref, b2n_ref,
                     z_ref, o1_ref, o2_ref):
    z1 = _assemble(w1_ref, seg0_ref, seg1_ref, ead0_ref, ead1_ref,
                   w5_ref, b5_ref, g_ref, be_ref, res_ref)
    z_ref[...] = z1
    o1_ref[...] = jnp.dot(z1, w1n_ref[...], preferred_element_type=jnp.float32) + b1n_ref[...]
    o2_ref[...] = jnp.dot(z1, w2n_ref[...], preferred_element_type=jnp.float32) + b2n_ref[...]


def _combine_body(w1_ref, seg0_ref, seg1_ref, ead0_ref, ead1_ref,
                  w5_ref, b5_ref, g_ref, be_ref, res_ref, z_ref):
    z_ref[...] = _assemble(w1_ref, seg0_ref, seg1_ref, ead0_ref, ead1_ref,
                           w5_ref, b5_ref, g_ref, be_ref, res_ref)


def _combine_mm(w1, seg, ead, W5, B5, G, Be, res, W1n, B1n, W2n, B2n):
    return pl.pallas_call(
        _combine_mm_body,
        grid=(GRID,),
        in_specs=[_row_spec, _p0_spec, _p1_spec, _p0_spec, _p1_spec,
                  _w5_spec, _b_spec, _b_spec, _b_spec, _row_spec,
                  _w_spec, _b_spec, _w_spec, _b_spec],
        out_specs=[_row_spec, _row_spec, _row_spec],
        out_shape=[jax.ShapeDtypeStruct((N, D), jnp.float32)] * 3,
    )(w1, seg, seg, ead, ead, W5, B5.reshape(1, D),
      G.reshape(1, D), Be.reshape(1, D), res, W1n, B1n.reshape(1, D),
      W2n, B2n.reshape(1, D))


def _combine(w1, seg, ead, W5, B5, G, Be, res):
    return pl.pallas_call(
        _combine_body,
        grid=(GRID,),
        in_specs=[_row_spec, _p0_spec, _p1_spec, _p0_spec, _p1_spec,
                  _w5_spec, _b_spec, _b_spec, _b_spec, _row_spec],
        out_specs=_row_spec,
        out_shape=jax.ShapeDtypeStruct((N, D), jnp.float32),
    )(w1, seg, seg, ead, ead, W5, B5.reshape(1, D),
      G.reshape(1, D), Be.reshape(1, D), res)


def kernel(x, edge_index, edge_attr,
           W1a, B1a, W2a, B2a, W3a, B3a, W4a, B4a, W5a, B5a, G1, Be1,
           W1b, B1b, W2b, B2b, W3b, B3b, W4b, B4b, W5b, B5b, G2, Be2):
    dst = edge_index[0]
    src = edge_index[1]

    ea128 = jnp.pad(
        jnp.concatenate([edge_attr, jnp.ones((E, 1), jnp.float32)], axis=1),
        ((0, 0), (0, D - DE - 1)))

    ead = _make_ea_deg_sc()(ea128, dst)
    w1a, w2a = _mm2(x, W1a, B1a, W2a, B2a)
    sega = _make_segsum_sc()(w2a, src, dst)
    z1, w1b, w2b = _combine_mm(w1a, sega, ead, W5a, B5a, G1, Be1, x,
                               W1b, B1b, W2b, B2b)
    segb = _make_segsum_sc()(w2b, src, dst)
    z2 = _combine(w1b, segb, ead, W5b, B5b, G2, Be2, z1)
    return z2
